# baseline (speedup 1.0000x reference)
import jax
import jax.numpy as jnp
from jax import lax
from jax.experimental import pallas as pl
from jax.experimental.pallas import tpu as pltpu
from jax.experimental.pallas import tpu_sc as plsc
import functools

_NUM_ITEMS = 100000
_HIST = 20
_BATCH = 1024
_NC = 2
_NS = 16
_L = 16
_NW = _NC * _NS
_ROWS_PER_W = _BATCH // _NW
_HP = 32
_NQ = 5
_Q = _NUM_ITEMS // _NQ


def _sc_body(uid_hbm, upi_hbm, p_hbm, out_hbm, uid_v, items_v, p_v, zbufs,
             gsem, *dsems):
    c = lax.axis_index("c")
    s = lax.axis_index("s")
    wid = s * _NC + c
    base = wid * _ROWS_PER_W

    pltpu.sync_copy(uid_hbm.at[pl.ds(base, _ROWS_PER_W)], uid_v)
    pltpu.async_copy(upi_hbm.at[uid_v], items_v, gsem).wait()
    pltpu.sync_copy(p_hbm, p_v)

    vval = p_v[...] + 100.0
    vzero = jnp.zeros((_L,), jnp.float32)

    def zfill(j, carry):
        for q in range(_NQ):
            zbufs[q, pl.ds(j * _L, _L)] = vzero
        return carry

    lax.fori_loop(0, _Q // _L, zfill, 0)

    mask_hi = lax.iota(jnp.int32, _L) >= 12

    def scat(q, idx0, idx1, val):
        lo = q * _Q
        for idxw, bm in ((idx0, None), (idx1, mask_hi)):
            m = (idxw >= lo) & (idxw < lo + _Q)
            if bm is not None:
                m = m & bm
            loc = jnp.where(m, idxw - lo, 0)
            plsc.store_scatter(zbufs.at[q], [loc], val, mask=m)

    def row_idx(r):
        return items_v[r, pl.ds(0, _L)], items_v[r, pl.ds(4, _L)]

    def issue(r, q):
        pltpu.async_copy(zbufs.at[q],
                         out_hbm.at[base + r, pl.ds(q * _Q, _Q)], dsems[q])

    def wait(r, q):
        pltpu.make_async_copy(zbufs.at[q],
                              out_hbm.at[base + r, pl.ds(q * _Q, _Q)],
                              dsems[q]).wait()

    idx0, idx1 = row_idx(0)
    for q in range(_NQ):
        scat(q, idx0, idx1, vval)
        issue(0, q)

    def row(r, carry):
        pidx0, pidx1 = row_idx(r - 1)
        idx0, idx1 = row_idx(r)
        for q in range(_NQ):
            wait(r - 1, q)
            scat(q, pidx0, pidx1, vzero)
            scat(q, idx0, idx1, vval)
            issue(r, q)
        return carry

    lax.fori_loop(1, _ROWS_PER_W, row, 0)

    for q in range(_NQ):
        wait(_ROWS_PER_W - 1, q)


@jax.jit
def kernel(users_ids, users_pos_items, param):
    mesh = plsc.VectorSubcoreMesh(
        core_axis_name="c", subcore_axis_name="s", num_cores=_NC,
        num_subcores=_NS)
    p16 = jnp.broadcast_to(param.astype(jnp.float32), (_L,))
    upi_p = jnp.pad(users_pos_items.astype(jnp.int32),
                    ((0, 0), (0, _HP - _HIST)))
    run = functools.partial(
        pl.kernel,
        out_type=jax.ShapeDtypeStruct((_BATCH, _NUM_ITEMS), jnp.float32),
        mesh=mesh,
        compiler_params=pltpu.CompilerParams(
            needs_layout_passes=False, use_tc_tiling_on_sc=False),
        scratch_types=[
            pltpu.VMEM((_ROWS_PER_W,), jnp.int32),
            pltpu.VMEM((_ROWS_PER_W, _HP), jnp.int32),
            pltpu.VMEM((_L,), jnp.float32),
            pltpu.VMEM((_NQ, _Q), jnp.float32),
            pltpu.SemaphoreType.DMA,
        ] + [pltpu.SemaphoreType.DMA] * _NQ,
    )(_sc_body)
    return run(users_ids.astype(jnp.int32), upi_p, p16)
